# baseline (speedup 1.0000x reference)
import functools

import jax
import jax.numpy as jnp
from jax import lax
from jax.experimental import pallas as pl
from jax.experimental.pallas import tpu as pltpu
from jax.experimental.pallas import tpu_sc as plsc

N_NODES = 50000
H = 4
F = 32
IN = 128
BLK = 256
ROWS = 50176
TPR = ROWS // 16
E_PAD = 262144
E_ROWS = E_PAD // 128

_SC_PARAMS = pltpu.CompilerParams(
    needs_layout_passes=False, use_tc_tiling_on_sc=False)

_MESH = plsc.VectorSubcoreMesh(core_axis_name="c", subcore_axis_name="s")


def _prep_body(feat_ref, w_ref, al_ref, ar_ref, heads_ref, el_ref, er_ref):
    i = pl.program_id(0)
    x = feat_ref[...]
    y = jnp.dot(x, w_ref[...], preferred_element_type=jnp.float32)
    row = i * BLK + lax.broadcasted_iota(jnp.int32, (BLK, 1), 0)
    y = jnp.where(row < N_NODES, y, 0.0)
    f_idx = lax.broadcasted_iota(jnp.int32, (IN, 16), 0)
    c_idx = lax.broadcasted_iota(jnp.int32, (IN, 16), 1)
    sel = ((f_idx // F) == c_idx).astype(jnp.float32)
    el_ref[...] = jnp.dot(y * al_ref[...], sel, preferred_element_type=jnp.float32)
    er_ref[...] = jnp.dot(y * ar_ref[...], sel, preferred_element_type=jnp.float32)
    for h in range(H):
        heads_ref[h] = y[:, h * F:(h + 1) * F]


def _tc_prep(feat, w_mat, al, ar):
    grid = ROWS // BLK
    return pl.pallas_call(
        _prep_body,
        grid=(grid,),
        in_specs=[pl.BlockSpec((BLK, IN), lambda i: (i, 0)),
                  pl.BlockSpec((IN, IN), lambda i: (0, 0)),
                  pl.BlockSpec((1, IN), lambda i: (0, 0)),
                  pl.BlockSpec((1, IN), lambda i: (0, 0))],
        out_specs=[pl.BlockSpec((H, BLK, F), lambda i: (0, i, 0)),
                   pl.BlockSpec((BLK, 16), lambda i: (i, 0)),
                   pl.BlockSpec((BLK, 16), lambda i: (i, 0))],
        out_shape=[jax.ShapeDtypeStruct((H, ROWS, F), jnp.float32),
                   jax.ShapeDtypeStruct((ROWS, 16), jnp.float32),
                   jax.ShapeDtypeStruct((ROWS, 16), jnp.float32)],
    )(feat, w_mat, al.reshape(1, IN), ar.reshape(1, IN))


def _sc_u1(el1, er1, s1, d1, el2, er2, s2, d2):
    @functools.partial(
        pl.kernel,
        out_type=[jax.ShapeDtypeStruct((H, E_ROWS, 128), jnp.float32),
                  jax.ShapeDtypeStruct((H, E_ROWS, 128), jnp.float32),
                  jax.ShapeDtypeStruct((2, ROWS, 16), jnp.float32)],
        mesh=_MESH,
        scratch_types=[pltpu.VMEM((8, 128), jnp.int32),
                       pltpu.VMEM((8, 128), jnp.int32),
                       pltpu.VMEM((1024, 16), jnp.float32),
                       pltpu.VMEM((1024, 16), jnp.float32),
                       pltpu.VMEM((1024, 16), jnp.float32),
                       pltpu.VMEM((H, 8, 128), jnp.float32),
                       pltpu.VMEM((392, 16), jnp.float32),
                       pltpu.VMEM_SHARED((ROWS, 16), jnp.float32),
                       pltpu.SemaphoreType.DMA],
        compiler_params=_SC_PARAMS,
    )
    def k(el1_h, er1_h, s1_h, d1_h, el2_h, er2_h, s2_h, d2_h,
          w1_out, w2_out, den_out,
          src_v, dst_v, el_r, er_r, w_v, wcols, zbuf, den_sh, sem):
        core = lax.axis_index("c")
        sub = lax.axis_index("s")

        @pl.loop(0, 392)
        def _z(i):
            zbuf[i] = jnp.zeros((16,), jnp.float32)

        @pl.loop(0, TPR // 392)
        def _zc(j):
            pltpu.sync_copy(zbuf, den_sh.at[pl.ds(sub * TPR + j * 392, 392)])

        plsc.subcore_barrier()

        def run_rel(el_hbm, er_hbm, src_hbm, dst_hbm, w_out, den_slot):
            @pl.loop(0, 16)
            def _chunk(ci):
                rb = sub * 128 + ci * 8
                pltpu.sync_copy(src_hbm.at[pl.ds(rb, 8)], src_v)
                pltpu.sync_copy(dst_hbm.at[pl.ds(rb, 8)], dst_v)
                cps = []
                for g in range(8):
                    cps.append(pltpu.async_copy(
                        el_hbm.at[src_v.at[g]], el_r.at[pl.ds(g * 128, 128)], sem))
                    cps.append(pltpu.async_copy(
                        er_hbm.at[dst_v.at[g]], er_r.at[pl.ds(g * 128, 128)], sem))
                for c in cps:
                    c.wait()

                @pl.loop(0, 1024, unroll=4)
                def _e(e):
                    v = el_r[e] + er_r[e]
                    w_v[e] = jnp.exp(jnp.maximum(v, v * jnp.float32(0.2)))

                for g in range(8):
                    pltpu.sync_copy(w_v.at[pl.ds(g * 128, 128)],
                                    den_sh.at[dst_v.at[g]], add=True)

                lanes = lax.iota(jnp.int32, 16)

                @pl.loop(0, 64, unroll=2)
                def _t(r16):
                    rows = r16 * 16 + lanes
                    r = r16 // 8
                    c0 = (r16 % 8) * 16
                    for h in range(H):
                        col = jnp.full((16,), h, jnp.int32)
                        wcols[h, r, pl.ds(c0, 16)] = plsc.load_gather(w_v, [rows, col])

                for h in range(H):
                    pltpu.sync_copy(wcols.at[h], w_out.at[h, pl.ds(rb, 8)])

            plsc.subcore_barrier()
            pltpu.sync_copy(den_sh.at[pl.ds(sub * TPR, TPR)],
                            den_out.at[den_slot, pl.ds(sub * TPR, TPR)])

        @pl.when(core == 0)
        def _():
            run_rel(el1_h, er1_h, s1_h, d1_h, w1_out, 0)

        @pl.when(core == 1)
        def _():
            run_rel(el2_h, er2_h, s2_h, d2_h, w2_out, 1)

    return k(el1, er1, s1, d1, el2, er2, s2, d2)


def _sc_u2(feat1, s1, d1, w1, feat2, s2, d2, w2):
    @functools.partial(
        pl.kernel,
        out_type=[jax.ShapeDtypeStruct((H, ROWS, F), jnp.float32),
                  jax.ShapeDtypeStruct((H, ROWS, F), jnp.float32)],
        mesh=_MESH,
        scratch_types=[pltpu.VMEM((4, 128), jnp.int32),
                       pltpu.VMEM((4, 128), jnp.int32),
                       pltpu.VMEM((4, 128), jnp.float32),
                       pltpu.VMEM((512, F), jnp.float32),
                       pltpu.VMEM_SHARED((ROWS, F), jnp.float32),
                       pltpu.SemaphoreType.DMA],
        compiler_params=_SC_PARAMS,
    )
    def k(f1_h, s1_h, d1_h, w1_h, f2_h, s2_h, d2_h, w2_h, acc1_out, acc2_out,
          src_v, dst_v, w_v, rows_v, acc_sh, sem):
        core = lax.axis_index("c")
        sub = lax.axis_index("s")

        def run_head(feat_hbm, src_hbm, dst_hbm, w_hbm, acc_out, h):
            @pl.loop(0, 392)
            def _z(i):
                rows_v[i, pl.ds(0, 16)] = jnp.zeros((16,), jnp.float32)
                rows_v[i, pl.ds(16, 16)] = jnp.zeros((16,), jnp.float32)

            @pl.loop(0, TPR // 392)
            def _zc(j):
                pltpu.sync_copy(rows_v.at[pl.ds(0, 392)],
                                acc_sh.at[pl.ds(sub * TPR + j * 392, 392)])

            plsc.subcore_barrier()

            @pl.loop(0, 32)
            def _chunk(ci):
                rb = sub * 128 + ci * 4
                pltpu.sync_copy(src_hbm.at[pl.ds(rb, 4)], src_v)
                pltpu.sync_copy(dst_hbm.at[pl.ds(rb, 4)], dst_v)
                pltpu.sync_copy(w_hbm.at[h, pl.ds(rb, 4)], w_v)

                @pl.loop(0, 4)
                def _g(g):
                    for kk in range(8):
                        src_v[g, pl.ds(kk * 16, 16)] = (
                            src_v[g, pl.ds(kk * 16, 16)] + h * ROWS)

                cps = []
                for g in range(4):
                    cps.append(pltpu.async_copy(
                        feat_hbm.at[src_v.at[g]],
                        rows_v.at[pl.ds(g * 128, 128)], sem))
                for c in cps:
                    c.wait()

                @pl.loop(0, 512, unroll=4)
                def _e(e):
                    hi = jnp.full((16,), e // 128, jnp.int32)
                    lo = jnp.full((16,), e % 128, jnp.int32)
                    wv = plsc.load_gather(w_v, [hi, lo])
                    rows_v[e, pl.ds(0, 16)] = rows_v[e, pl.ds(0, 16)] * wv
                    rows_v[e, pl.ds(16, 16)] = rows_v[e, pl.ds(16, 16)] * wv

                for g in range(4):
                    pltpu.sync_copy(rows_v.at[pl.ds(g * 128, 128)],
                                    acc_sh.at[dst_v.at[g]], add=True)

            plsc.subcore_barrier()
            pltpu.sync_copy(acc_sh.at[pl.ds(sub * TPR, TPR)],
                            acc_out.at[h, pl.ds(sub * TPR, TPR)])
            plsc.subcore_barrier()

        @pl.when(core == 0)
        def _():
            for h in range(H):
                run_head(f1_h, s1_h, d1_h, w1_h, acc1_out, h)

        @pl.when(core == 1)
        def _():
            for h in range(H):
                run_head(f2_h, s2_h, d2_h, w2_h, acc2_out, h)

    return k(feat1, s1, d1, w1, feat2, s2, d2, w2)


def _fin_body(a1_ref, a2_ref, d_ref, b_ref, o_ref):
    d1 = d_ref[0]
    d2 = d_ref[1]
    parts = []
    for h in range(H):
        n1 = a1_ref[h] / jnp.maximum(d1[:, h:h + 1], 1e-20)
        n2 = a2_ref[h] / jnp.maximum(d2[:, h:h + 1], 1e-20)
        parts.append(n1 + n2)
    o_ref[...] = jnp.concatenate(parts, axis=1) + b_ref[...]


def _tc_fin(acc1, acc2, den, bias_sum):
    grid = ROWS // BLK
    return pl.pallas_call(
        _fin_body,
        grid=(grid,),
        in_specs=[pl.BlockSpec((H, BLK, F), lambda i: (0, i, 0)),
                  pl.BlockSpec((H, BLK, F), lambda i: (0, i, 0)),
                  pl.BlockSpec((2, BLK, 16), lambda i: (0, i, 0)),
                  pl.BlockSpec((1, IN), lambda i: (0, 0))],
        out_specs=pl.BlockSpec((BLK, IN), lambda i: (i, 0)),
        out_shape=jax.ShapeDtypeStruct((N_NODES, IN), jnp.float32),
    )(acc1, acc2, den, bias_sum)


def _pad_edges(ei):
    pad = E_PAD - ei.shape[1]
    src = jnp.concatenate([ei[0], jnp.full((pad,), N_NODES, jnp.int32)])
    dst = jnp.concatenate([ei[1], jnp.full((pad,), N_NODES, jnp.int32)])
    return src.reshape(E_ROWS, 128), dst.reshape(E_ROWS, 128)


def kernel(feat_item, feat_t, edge_index_i2t, edge_index_t2t,
           W_i2t, attn_l_i2t, attn_r_i2t, bias_i2t,
           W_t2t, attn_l_t2t, attn_r_t2t, bias_t2t):
    heads_i2t, el_i2t, _ = _tc_prep(feat_item, W_i2t, attn_l_i2t, attn_r_i2t)
    _, _, er_i2t = _tc_prep(feat_t, W_i2t, attn_l_i2t, attn_r_i2t)
    heads_t2t, el_t2t, er_t2t = _tc_prep(feat_t, W_t2t, attn_l_t2t, attn_r_t2t)

    s1, d1 = _pad_edges(edge_index_i2t)
    s2, d2 = _pad_edges(edge_index_t2t)

    w1, w2, den = _sc_u1(el_i2t, er_i2t, s1, d1, el_t2t, er_t2t, s2, d2)

    acc1, acc2 = _sc_u2(heads_i2t.reshape(H * ROWS, F), s1, d1, w1,
                        heads_t2t.reshape(H * ROWS, F), s2, d2, w2)

    out = _tc_fin(acc1, acc2, den, (bias_i2t + bias_t2t).reshape(1, IN))
    return out.reshape(N_NODES, H, F)


# speedup vs baseline: 1.1736x; 1.1736x over previous
import functools

import jax
import jax.numpy as jnp
from jax import lax
from jax.experimental import pallas as pl
from jax.experimental.pallas import tpu as pltpu
from jax.experimental.pallas import tpu_sc as plsc

N_NODES = 50000
H = 4
F = 32
IN = 128
BLK = 256
ROWS = 50176
TPR = ROWS // 16
E_PAD = 262144
E_ROWS = E_PAD // 128

_SC_PARAMS = pltpu.CompilerParams(
    needs_layout_passes=False, use_tc_tiling_on_sc=False)

_MESH = plsc.VectorSubcoreMesh(core_axis_name="c", subcore_axis_name="s")


def _prep_body(feat_ref, w_ref, al_ref, ar_ref, heads_ref, el_ref, er_ref):
    i = pl.program_id(0)
    x = feat_ref[...]
    y = jnp.dot(x, w_ref[...], preferred_element_type=jnp.float32)
    row = i * BLK + lax.broadcasted_iota(jnp.int32, (BLK, 1), 0)
    y = jnp.where(row < N_NODES, y, 0.0)
    f_idx = lax.broadcasted_iota(jnp.int32, (IN, 16), 0)
    c_idx = lax.broadcasted_iota(jnp.int32, (IN, 16), 1)
    sel = ((f_idx // F) == c_idx).astype(jnp.float32)
    el_ref[...] = jnp.dot(y * al_ref[...], sel, preferred_element_type=jnp.float32)
    er_ref[...] = jnp.dot(y * ar_ref[...], sel, preferred_element_type=jnp.float32)
    for h in range(H):
        heads_ref[h] = y[:, h * F:(h + 1) * F]


def _tc_prep(feat, w_mat, al, ar):
    grid = ROWS // BLK
    return pl.pallas_call(
        _prep_body,
        grid=(grid,),
        in_specs=[pl.BlockSpec((BLK, IN), lambda i: (i, 0)),
                  pl.BlockSpec((IN, IN), lambda i: (0, 0)),
                  pl.BlockSpec((1, IN), lambda i: (0, 0)),
                  pl.BlockSpec((1, IN), lambda i: (0, 0))],
        out_specs=[pl.BlockSpec((H, BLK, F), lambda i: (0, i, 0)),
                   pl.BlockSpec((BLK, 16), lambda i: (i, 0)),
                   pl.BlockSpec((BLK, 16), lambda i: (i, 0))],
        out_shape=[jax.ShapeDtypeStruct((H, ROWS, F), jnp.float32),
                   jax.ShapeDtypeStruct((ROWS, 16), jnp.float32),
                   jax.ShapeDtypeStruct((ROWS, 16), jnp.float32)],
    )(feat, w_mat, al.reshape(1, IN), ar.reshape(1, IN))


def _sc_u1(el1, er1, s1, d1, el2, er2, s2, d2):
    @functools.partial(
        pl.kernel,
        out_type=[jax.ShapeDtypeStruct((H, E_PAD), jnp.float32),
                  jax.ShapeDtypeStruct((H, E_PAD), jnp.float32),
                  jax.ShapeDtypeStruct((2, ROWS, 16), jnp.float32)],
        mesh=_MESH,
        scratch_types=[pltpu.VMEM((8, 128), jnp.int32),
                       pltpu.VMEM((8, 128), jnp.int32),
                       pltpu.VMEM((1024, 16), jnp.float32),
                       pltpu.VMEM((1024, 16), jnp.float32),
                       pltpu.VMEM((1024, 16), jnp.float32),
                       pltpu.VMEM((H, 1024), jnp.float32),
                       pltpu.VMEM((392, 16), jnp.float32),
                       pltpu.VMEM_SHARED((ROWS, 16), jnp.float32),
                       pltpu.SemaphoreType.DMA],
        compiler_params=_SC_PARAMS,
    )
    def k(el1_h, er1_h, s1_h, d1_h, el2_h, er2_h, s2_h, d2_h,
          w1_out, w2_out, den_out,
          src_v, dst_v, el_r, er_r, w_v, wcols, zbuf, den_sh, sem):
        core = lax.axis_index("c")
        sub = lax.axis_index("s")

        @pl.loop(0, 392)
        def _z(i):
            zbuf[i] = jnp.zeros((16,), jnp.float32)

        @pl.loop(0, TPR // 392)
        def _zc(j):
            pltpu.sync_copy(zbuf, den_sh.at[pl.ds(sub * TPR + j * 392, 392)])

        plsc.subcore_barrier()

        def run_rel(el_hbm, er_hbm, src_hbm, dst_hbm, w_out, den_slot):
            @pl.loop(0, 16)
            def _chunk(ci):
                rb = sub * 128 + ci * 8
                pltpu.sync_copy(src_hbm.at[pl.ds(rb, 8)], src_v)
                pltpu.sync_copy(dst_hbm.at[pl.ds(rb, 8)], dst_v)
                cps = []
                for g in range(8):
                    cps.append(pltpu.async_copy(
                        el_hbm.at[src_v.at[g]], el_r.at[pl.ds(g * 128, 128)], sem))
                    cps.append(pltpu.async_copy(
                        er_hbm.at[dst_v.at[g]], er_r.at[pl.ds(g * 128, 128)], sem))
                for c in cps:
                    c.wait()

                @pl.loop(0, 1024, unroll=4)
                def _e(e):
                    v = el_r[e] + er_r[e]
                    w_v[e] = jnp.exp(jnp.maximum(v, v * jnp.float32(0.2)))

                for g in range(8):
                    pltpu.sync_copy(w_v.at[pl.ds(g * 128, 128)],
                                    den_sh.at[dst_v.at[g]], add=True)

                lanes = lax.iota(jnp.int32, 16)

                @pl.loop(0, 64, unroll=2)
                def _t(r16):
                    rows = r16 * 16 + lanes
                    c0 = r16 * 16
                    for h in range(H):
                        col = jnp.full((16,), h, jnp.int32)
                        wcols[h, pl.ds(c0, 16)] = plsc.load_gather(w_v, [rows, col])

                for h in range(H):
                    pltpu.sync_copy(wcols.at[h], w_out.at[h, pl.ds(rb * 128, 1024)])

            plsc.subcore_barrier()
            pltpu.sync_copy(den_sh.at[pl.ds(sub * TPR, TPR)],
                            den_out.at[den_slot, pl.ds(sub * TPR, TPR)])

        @pl.when(core == 0)
        def _():
            run_rel(el1_h, er1_h, s1_h, d1_h, w1_out, 0)

        @pl.when(core == 1)
        def _():
            run_rel(el2_h, er2_h, s2_h, d2_h, w2_out, 1)

    return k(el1, er1, s1, d1, el2, er2, s2, d2)


def _sc_u2(feat1, s1, d1, w1, feat2, s2, d2, w2):
    @functools.partial(
        pl.kernel,
        out_type=[jax.ShapeDtypeStruct((H, ROWS, F), jnp.float32),
                  jax.ShapeDtypeStruct((H, ROWS, F), jnp.float32)],
        mesh=_MESH,
        scratch_types=[pltpu.VMEM((2, 128), jnp.int32),
                       pltpu.VMEM((2, 128), jnp.int32),
                       pltpu.VMEM((2, 128), jnp.int32),
                       pltpu.VMEM((2, 128), jnp.int32),
                       pltpu.VMEM((256,), jnp.float32),
                       pltpu.VMEM((256,), jnp.float32),
                       pltpu.VMEM((256, F), jnp.float32),
                       pltpu.VMEM((256, F), jnp.float32),
                       pltpu.SemaphoreType.DMA,
                       pltpu.SemaphoreType.DMA,
                       pltpu.SemaphoreType.DMA,
                       pltpu.SemaphoreType.DMA,
                       pltpu.SemaphoreType.DMA,
                       pltpu.SemaphoreType.DMA,
                       pltpu.SemaphoreType.DMA,
                       pltpu.SemaphoreType.DMA,
                       pltpu.VMEM_SHARED((ROWS, F), jnp.float32)],
        compiler_params=_SC_PARAMS,
    )
    def k(f1_h, s1_h, d1_h, w1_h, f2_h, s2_h, d2_h, w2_h, acc1_out, acc2_out,
          srcb0, srcb1, dstb0, dstb1, wb0, wb1, rows0, rows1,
          semi0, semi1, semd0, semd1, semg0, semg1, sems0, sems1, acc_sh):
        core = lax.axis_index("c")
        sub = lax.axis_index("s")
        srcb = (srcb0, srcb1)
        dstb = (dstb0, dstb1)
        wb = (wb0, wb1)
        rows = (rows0, rows1)
        semi = (semi0, semi1)
        semd = (semd0, semd1)
        semg = (semg0, semg1)
        sems = (sems0, sems1)

        def run_rel(feat_hbm, src_hbm, dst_hbm, w_hbm, acc_out):
            def fire_srcw(c, b, h):
                rbc = sub * 128 + c * 2
                pltpu.async_copy(src_hbm.at[pl.ds(rbc, 2)], srcb[b], semi[b])
                pltpu.async_copy(w_hbm.at[h, pl.ds(rbc * 128, 256)], wb[b], semi[b])

            def fire_dst(c, b):
                rbc = sub * 128 + c * 2
                pltpu.async_copy(dst_hbm.at[pl.ds(rbc, 2)], dstb[b], semd[b])

            def drain_srcw(b):
                pltpu.make_async_copy(src_hbm.at[pl.ds(0, 2)], srcb[b], semi[b]).wait()
                pltpu.make_async_copy(w_hbm.at[0, pl.ds(0, 256)], wb[b], semi[b]).wait()

            def drain_dst(b):
                pltpu.make_async_copy(dst_hbm.at[pl.ds(0, 2)], dstb[b], semd[b]).wait()

            def gidx(b, h):
                off = h * ROWS
                for g in range(2):
                    for kk in range(8):
                        srcb[b][g, pl.ds(kk * 16, 16)] = (
                            srcb[b][g, pl.ds(kk * 16, 16)] + off)

            def fire_gathers(b):
                for g in range(2):
                    pltpu.async_copy(feat_hbm.at[srcb[b].at[g]],
                                     rows[b].at[pl.ds(g * 128, 128)], semg[b])

            def drain_gathers(b):
                pltpu.make_async_copy(feat_hbm.at[pl.ds(0, 256)], rows[b],
                                      semg[b]).wait()

            def scale(b):
                @pl.loop(0, 256, unroll=4)
                def _e(e):
                    wv = plsc.load_gather(wb[b], [jnp.full((16,), e, jnp.int32)])
                    rows[b][e, pl.ds(0, 16)] = rows[b][e, pl.ds(0, 16)] * wv
                    rows[b][e, pl.ds(16, 16)] = rows[b][e, pl.ds(16, 16)] * wv

            def fire_scatters(b):
                for g in range(2):
                    pltpu.async_copy(rows[b].at[pl.ds(g * 128, 128)],
                                     acc_sh.at[dstb[b].at[g]], sems[b], add=True)

            def drain_scatters(b):
                pltpu.make_async_copy(feat_hbm.at[pl.ds(0, 256)], rows[b],
                                      sems[b]).wait()

            for h in range(H):
                @pl.loop(0, 196)
                def _z(i):
                    rows0[i, pl.ds(0, 16)] = jnp.zeros((16,), jnp.float32)
                    rows0[i, pl.ds(16, 16)] = jnp.zeros((16,), jnp.float32)

                @pl.loop(0, TPR // 196)
                def _zc(j):
                    pltpu.sync_copy(rows0.at[pl.ds(0, 196)],
                                    acc_sh.at[pl.ds(sub * TPR + j * 196, 196)])

                plsc.subcore_barrier()

                fire_srcw(0, 0, h)
                fire_dst(0, 0)
                fire_srcw(1, 1, h)
                fire_dst(1, 1)

                @pl.loop(0, 32)
                def _pair(pi):
                    c0 = pi * 2
                    drain_srcw(0)
                    gidx(0, h)

                    @pl.when(pi > 0)
                    def _():
                        drain_scatters(0)
                        fire_dst(c0, 0)

                    fire_gathers(0)

                    drain_srcw(1)
                    gidx(1, h)

                    @pl.when(pi > 0)
                    def _():
                        drain_scatters(1)
                        fire_dst(c0 + 1, 1)

                    fire_gathers(1)

                    drain_dst(0)
                    drain_gathers(0)
                    scale(0)
                    fire_scatters(0)

                    @pl.when(pi < 31)
                    def _():
                        fire_srcw(c0 + 2, 0, h)

                    drain_dst(1)
                    drain_gathers(1)
                    scale(1)
                    fire_scatters(1)

                    @pl.when(pi < 31)
                    def _():
                        fire_srcw(c0 + 3, 1, h)

                drain_scatters(0)
                drain_scatters(1)

                plsc.subcore_barrier()
                pltpu.sync_copy(acc_sh.at[pl.ds(sub * TPR, TPR)],
                                acc_out.at[h, pl.ds(sub * TPR, TPR)])
                plsc.subcore_barrier()

        @pl.when(core == 0)
        def _():
            run_rel(f1_h, s1_h, d1_h, w1_h, acc1_out)

        @pl.when(core == 1)
        def _():
            run_rel(f2_h, s2_h, d2_h, w2_h, acc2_out)

    return k(feat1, s1, d1, w1, feat2, s2, d2, w2)


def _fin_body(a1_ref, a2_ref, d_ref, b_ref, o_ref):
    d1 = d_ref[0]
    d2 = d_ref[1]
    parts = []
    for h in range(H):
        n1 = a1_ref[h] / jnp.maximum(d1[:, h:h + 1], 1e-20)
        n2 = a2_ref[h] / jnp.maximum(d2[:, h:h + 1], 1e-20)
        parts.append(n1 + n2)
    o_ref[...] = jnp.concatenate(parts, axis=1) + b_ref[...]


def _tc_fin(acc1, acc2, den, bias_sum):
    grid = ROWS // BLK
    return pl.pallas_call(
        _fin_body,
        grid=(grid,),
        in_specs=[pl.BlockSpec((H, BLK, F), lambda i: (0, i, 0)),
                  pl.BlockSpec((H, BLK, F), lambda i: (0, i, 0)),
                  pl.BlockSpec((2, BLK, 16), lambda i: (0, i, 0)),
                  pl.BlockSpec((1, IN), lambda i: (0, 0))],
        out_specs=pl.BlockSpec((BLK, IN), lambda i: (i, 0)),
        out_shape=jax.ShapeDtypeStruct((N_NODES, IN), jnp.float32),
    )(acc1, acc2, den, bias_sum)


def _pad_edges(ei):
    pad = E_PAD - ei.shape[1]
    src = jnp.concatenate([ei[0], jnp.full((pad,), N_NODES, jnp.int32)])
    dst = jnp.concatenate([ei[1], jnp.full((pad,), N_NODES, jnp.int32)])
    return src.reshape(E_ROWS, 128), dst.reshape(E_ROWS, 128)


def kernel(feat_item, feat_t, edge_index_i2t, edge_index_t2t,
           W_i2t, attn_l_i2t, attn_r_i2t, bias_i2t,
           W_t2t, attn_l_t2t, attn_r_t2t, bias_t2t):
    heads_i2t, el_i2t, _ = _tc_prep(feat_item, W_i2t, attn_l_i2t, attn_r_i2t)
    _, _, er_i2t = _tc_prep(feat_t, W_i2t, attn_l_i2t, attn_r_i2t)
    heads_t2t, el_t2t, er_t2t = _tc_prep(feat_t, W_t2t, attn_l_t2t, attn_r_t2t)

    s1, d1 = _pad_edges(edge_index_i2t)
    s2, d2 = _pad_edges(edge_index_t2t)

    w1, w2, den = _sc_u1(el_i2t, er_i2t, s1, d1, el_t2t, er_t2t, s2, d2)

    acc1, acc2 = _sc_u2(heads_i2t.reshape(H * ROWS, F), s1, d1, w1,
                        heads_t2t.reshape(H * ROWS, F), s2, d2, w2)

    out = _tc_fin(acc1, acc2, den, (bias_i2t + bias_t2t).reshape(1, IN))
    return out.reshape(N_NODES, H, F)


# speedup vs baseline: 1.2683x; 1.0807x over previous
import functools

import jax
import jax.numpy as jnp
from jax import lax
from jax.experimental import pallas as pl
from jax.experimental.pallas import tpu as pltpu
from jax.experimental.pallas import tpu_sc as plsc

N_NODES = 50000
H = 4
F = 32
IN = 128
BLK = 256
ROWS = 50176
TPR = ROWS // 16
E_PAD = 262144
E_ROWS = E_PAD // 128

_SC_PARAMS = pltpu.CompilerParams(
    needs_layout_passes=False, use_tc_tiling_on_sc=False)

_MESH = plsc.VectorSubcoreMesh(core_axis_name="c", subcore_axis_name="s")


def _prep_body(feat_ref, w_ref, al_ref, ar_ref, heads_ref, el_ref, er_ref):
    i = pl.program_id(0)
    x = feat_ref[...]
    y = jnp.dot(x, w_ref[...], preferred_element_type=jnp.float32)
    row = i * BLK + lax.broadcasted_iota(jnp.int32, (BLK, 1), 0)
    y = jnp.where(row < N_NODES, y, 0.0)
    f_idx = lax.broadcasted_iota(jnp.int32, (IN, 16), 0)
    c_idx = lax.broadcasted_iota(jnp.int32, (IN, 16), 1)
    sel = ((f_idx // F) == c_idx).astype(jnp.float32)
    el_ref[...] = jnp.dot(y * al_ref[...], sel, preferred_element_type=jnp.float32)
    er_ref[...] = jnp.dot(y * ar_ref[...], sel, preferred_element_type=jnp.float32)
    for h in range(H):
        heads_ref[h] = y[:, h * F:(h + 1) * F]


def _tc_prep(feat, w_mat, al, ar):
    grid = ROWS // BLK
    return pl.pallas_call(
        _prep_body,
        grid=(grid,),
        in_specs=[pl.BlockSpec((BLK, IN), lambda i: (i, 0)),
                  pl.BlockSpec((IN, IN), lambda i: (0, 0)),
                  pl.BlockSpec((1, IN), lambda i: (0, 0)),
                  pl.BlockSpec((1, IN), lambda i: (0, 0))],
        out_specs=[pl.BlockSpec((H, BLK, F), lambda i: (0, i, 0)),
                   pl.BlockSpec((BLK, 16), lambda i: (i, 0)),
                   pl.BlockSpec((BLK, 16), lambda i: (i, 0))],
        out_shape=[jax.ShapeDtypeStruct((H, ROWS, F), jnp.float32),
                   jax.ShapeDtypeStruct((ROWS, 16), jnp.float32),
                   jax.ShapeDtypeStruct((ROWS, 16), jnp.float32)],
    )(feat, w_mat, al.reshape(1, IN), ar.reshape(1, IN))


def _sc_u1(el1, er1, s1, d1, el2, er2, s2, d2):
    @functools.partial(
        pl.kernel,
        out_type=[jax.ShapeDtypeStruct((H, E_PAD), jnp.float32),
                  jax.ShapeDtypeStruct((H, E_PAD), jnp.float32),
                  jax.ShapeDtypeStruct((2, ROWS, 16), jnp.float32)],
        mesh=_MESH,
        scratch_types=[pltpu.VMEM((8, 128), jnp.int32),
                       pltpu.VMEM((8, 128), jnp.int32),
                       pltpu.VMEM((1024, 16), jnp.float32),
                       pltpu.VMEM((1024, 16), jnp.float32),
                       pltpu.VMEM((1024, 16), jnp.float32),
                       pltpu.VMEM((H, 1024), jnp.float32),
                       pltpu.VMEM((392, 16), jnp.float32),
                       pltpu.VMEM_SHARED((ROWS, 16), jnp.float32),
                       pltpu.SemaphoreType.DMA],
        compiler_params=_SC_PARAMS,
    )
    def k(el1_h, er1_h, s1_h, d1_h, el2_h, er2_h, s2_h, d2_h,
          w1_out, w2_out, den_out,
          src_v, dst_v, el_r, er_r, w_v, wcols, zbuf, den_sh, sem):
        core = lax.axis_index("c")
        sub = lax.axis_index("s")

        @pl.loop(0, 392)
        def _z(i):
            zbuf[i] = jnp.zeros((16,), jnp.float32)

        @pl.loop(0, TPR // 392)
        def _zc(j):
            pltpu.sync_copy(zbuf, den_sh.at[pl.ds(sub * TPR + j * 392, 392)])

        plsc.subcore_barrier()

        def run_rel(el_hbm, er_hbm, src_hbm, dst_hbm, w_out, den_slot):
            @pl.loop(0, 16)
            def _chunk(ci):
                rb = sub * 128 + ci * 8
                pltpu.sync_copy(src_hbm.at[pl.ds(rb, 8)], src_v)
                pltpu.sync_copy(dst_hbm.at[pl.ds(rb, 8)], dst_v)
                cps = []
                for g in range(8):
                    cps.append(pltpu.async_copy(
                        el_hbm.at[src_v.at[g]], el_r.at[pl.ds(g * 128, 128)], sem))
                    cps.append(pltpu.async_copy(
                        er_hbm.at[dst_v.at[g]], er_r.at[pl.ds(g * 128, 128)], sem))
                for c in cps:
                    c.wait()

                @pl.loop(0, 1024, unroll=4)
                def _e(e):
                    v = el_r[e] + er_r[e]
                    w_v[e] = jnp.exp(jnp.maximum(v, v * jnp.float32(0.2)))

                for g in range(8):
                    pltpu.sync_copy(w_v.at[pl.ds(g * 128, 128)],
                                    den_sh.at[dst_v.at[g]], add=True)

                lanes = lax.iota(jnp.int32, 16)

                @pl.loop(0, 64, unroll=2)
                def _t(r16):
                    rows = r16 * 16 + lanes
                    c0 = r16 * 16
                    for h in range(H):
                        col = jnp.full((16,), h, jnp.int32)
                        wcols[h, pl.ds(c0, 16)] = plsc.load_gather(w_v, [rows, col])

                for h in range(H):
                    pltpu.sync_copy(wcols.at[h], w_out.at[h, pl.ds(rb * 128, 1024)])

            plsc.subcore_barrier()
            pltpu.sync_copy(den_sh.at[pl.ds(sub * TPR, TPR)],
                            den_out.at[den_slot, pl.ds(sub * TPR, TPR)])

        @pl.when(core == 0)
        def _():
            run_rel(el1_h, er1_h, s1_h, d1_h, w1_out, 0)

        @pl.when(core == 1)
        def _():
            run_rel(el2_h, er2_h, s2_h, d2_h, w2_out, 1)

    return k(el1, er1, s1, d1, el2, er2, s2, d2)


def _sc_u2(feat1, s1, d1, w1, feat2, s2, d2, w2):
    @functools.partial(
        pl.kernel,
        out_type=[jax.ShapeDtypeStruct((H, ROWS, F), jnp.float32),
                  jax.ShapeDtypeStruct((H, ROWS, F), jnp.float32)],
        mesh=_MESH,
        scratch_types=([pltpu.VMEM((2, 128), jnp.int32)] * 3
                       + [pltpu.VMEM((2, 128), jnp.int32)] * 3
                       + [pltpu.VMEM((256,), jnp.float32)] * 3
                       + [pltpu.VMEM((256, F), jnp.float32)] * 3
                       + [pltpu.SemaphoreType.DMA] * 12
                       + [pltpu.VMEM_SHARED((ROWS, F), jnp.float32)]),
        compiler_params=_SC_PARAMS,
    )
    def k(f1_h, s1_h, d1_h, w1_h, f2_h, s2_h, d2_h, w2_h, acc1_out, acc2_out,
          srcb0, srcb1, srcb2, dstb0, dstb1, dstb2, wb0, wb1, wb2,
          rows0, rows1, rows2,
          semi0, semi1, semi2, semd0, semd1, semd2,
          semg0, semg1, semg2, sems0, sems1, sems2, acc_sh):
        core = lax.axis_index("c")
        sub = lax.axis_index("s")
        srcb = (srcb0, srcb1, srcb2)
        dstb = (dstb0, dstb1, dstb2)
        wb = (wb0, wb1, wb2)
        rows = (rows0, rows1, rows2)
        semi = (semi0, semi1, semi2)
        semd = (semd0, semd1, semd2)
        semg = (semg0, semg1, semg2)
        sems = (sems0, sems1, sems2)

        def run_rel(feat_hbm, src_hbm, dst_hbm, w_hbm, acc_out):
            def fire_idx(c, b, h):
                rbc = sub * 128 + c * 2
                pltpu.async_copy(src_hbm.at[pl.ds(rbc, 2)], srcb[b], semi[b])
                pltpu.async_copy(w_hbm.at[h, pl.ds(rbc * 128, 256)], wb[b], semi[b])
                pltpu.async_copy(dst_hbm.at[pl.ds(rbc, 2)], dstb[b], semd[b])

            def drain_idx(b):
                pltpu.make_async_copy(src_hbm.at[pl.ds(0, 2)], srcb[b], semi[b]).wait()
                pltpu.make_async_copy(w_hbm.at[0, pl.ds(0, 256)], wb[b], semi[b]).wait()
                pltpu.make_async_copy(dst_hbm.at[pl.ds(0, 2)], dstb[b], semd[b]).wait()

            def gidx(b, h):
                off = h * ROWS
                for g in range(2):
                    for kk in range(8):
                        srcb[b][g, pl.ds(kk * 16, 16)] = (
                            srcb[b][g, pl.ds(kk * 16, 16)] + off)

            def fire_gathers(b):
                for g in range(2):
                    pltpu.async_copy(feat_hbm.at[srcb[b].at[g]],
                                     rows[b].at[pl.ds(g * 128, 128)], semg[b])

            def drain_gathers(b):
                pltpu.make_async_copy(feat_hbm.at[pl.ds(0, 256)], rows[b],
                                      semg[b]).wait()

            def scale(b):
                @pl.loop(0, 256, unroll=4)
                def _e(e):
                    wv = plsc.load_gather(wb[b], [jnp.full((16,), e, jnp.int32)])
                    rows[b][e, pl.ds(0, 16)] = rows[b][e, pl.ds(0, 16)] * wv
                    rows[b][e, pl.ds(16, 16)] = rows[b][e, pl.ds(16, 16)] * wv

            def fire_scatters(b):
                for g in range(2):
                    pltpu.async_copy(rows[b].at[pl.ds(g * 128, 128)],
                                     acc_sh.at[dstb[b].at[g]], sems[b], add=True)

            def drain_scatters(b):
                pltpu.make_async_copy(feat_hbm.at[pl.ds(0, 256)], rows[b],
                                      sems[b]).wait()

            for h in range(H):
                @pl.loop(0, 196)
                def _z(i):
                    rows0[i, pl.ds(0, 16)] = jnp.zeros((16,), jnp.float32)
                    rows0[i, pl.ds(16, 16)] = jnp.zeros((16,), jnp.float32)

                @pl.loop(0, TPR // 196)
                def _zc(j):
                    pltpu.sync_copy(rows0.at[pl.ds(0, 196)],
                                    acc_sh.at[pl.ds(sub * TPR + j * 196, 196)])

                plsc.subcore_barrier()

                fire_idx(0, 0, h)
                drain_idx(0)
                gidx(0, h)
                fire_gathers(0)
                fire_idx(1, 1, h)
                drain_idx(1)
                gidx(1, h)
                fire_gathers(1)

                def step(t, j, k):
                    b0 = k % 3
                    b2 = (k + 2) % 3

                    def stage_a():
                        drain_scatters(b2)

                    if k == 0:
                        @pl.when(j > 0)
                        def _():
                            stage_a()
                    else:
                        stage_a()

                    def stage_be():
                        fire_idx(t + 2, b2, h)
                        drain_gathers(b0)
                        scale(b0)
                        fire_scatters(b0)
                        drain_idx(b2)
                        gidx(b2, h)
                        fire_gathers(b2)

                    def stage_c_only():
                        drain_gathers(b0)
                        scale(b0)
                        fire_scatters(b0)

                    if k == 2:
                        @pl.when(j < 20)
                        def _():
                            stage_be()

                        @pl.when(j >= 20)
                        def _():
                            stage_c_only()
                    else:
                        stage_be()

                @pl.loop(0, 21)
                def _triple(j):
                    for k in range(3):
                        step(3 * j + k, j, k)

                drain_scatters(2)
                drain_gathers(0)
                scale(0)
                fire_scatters(0)
                drain_scatters(0)

                plsc.subcore_barrier()
                pltpu.sync_copy(acc_sh.at[pl.ds(sub * TPR, TPR)],
                                acc_out.at[h, pl.ds(sub * TPR, TPR)])
                plsc.subcore_barrier()

        @pl.when(core == 0)
        def _():
            run_rel(f1_h, s1_h, d1_h, w1_h, acc1_out)

        @pl.when(core == 1)
        def _():
            run_rel(f2_h, s2_h, d2_h, w2_h, acc2_out)

    return k(feat1, s1, d1, w1, feat2, s2, d2, w2)


def _fin_body(a1_ref, a2_ref, d_ref, b_ref, o_ref):
    d1 = d_ref[0]
    d2 = d_ref[1]
    parts = []
    for h in range(H):
        n1 = a1_ref[h] / jnp.maximum(d1[:, h:h + 1], 1e-20)
        n2 = a2_ref[h] / jnp.maximum(d2[:, h:h + 1], 1e-20)
        parts.append(n1 + n2)
    o_ref[...] = jnp.concatenate(parts, axis=1) + b_ref[...]


def _tc_fin(acc1, acc2, den, bias_sum):
    grid = ROWS // BLK
    return pl.pallas_call(
        _fin_body,
        grid=(grid,),
        in_specs=[pl.BlockSpec((H, BLK, F), lambda i: (0, i, 0)),
                  pl.BlockSpec((H, BLK, F), lambda i: (0, i, 0)),
                  pl.BlockSpec((2, BLK, 16), lambda i: (0, i, 0)),
                  pl.BlockSpec((1, IN), lambda i: (0, 0))],
        out_specs=pl.BlockSpec((BLK, IN), lambda i: (i, 0)),
        out_shape=jax.ShapeDtypeStruct((N_NODES, IN), jnp.float32),
    )(acc1, acc2, den, bias_sum)


def _pad_edges(ei):
    pad = E_PAD - ei.shape[1]
    src = jnp.concatenate([ei[0], jnp.full((pad,), N_NODES, jnp.int32)])
    dst = jnp.concatenate([ei[1], jnp.full((pad,), N_NODES, jnp.int32)])
    return src.reshape(E_ROWS, 128), dst.reshape(E_ROWS, 128)


def kernel(feat_item, feat_t, edge_index_i2t, edge_index_t2t,
           W_i2t, attn_l_i2t, attn_r_i2t, bias_i2t,
           W_t2t, attn_l_t2t, attn_r_t2t, bias_t2t):
    heads_i2t, el_i2t, _ = _tc_prep(feat_item, W_i2t, attn_l_i2t, attn_r_i2t)
    _, _, er_i2t = _tc_prep(feat_t, W_i2t, attn_l_i2t, attn_r_i2t)
    heads_t2t, el_t2t, er_t2t = _tc_prep(feat_t, W_t2t, attn_l_t2t, attn_r_t2t)

    s1, d1 = _pad_edges(edge_index_i2t)
    s2, d2 = _pad_edges(edge_index_t2t)

    w1, w2, den = _sc_u1(el_i2t, er_i2t, s1, d1, el_t2t, er_t2t, s2, d2)

    acc1, acc2 = _sc_u2(heads_i2t.reshape(H * ROWS, F), s1, d1, w1,
                        heads_t2t.reshape(H * ROWS, F), s2, d2, w2)

    out = _tc_fin(acc1, acc2, den, (bias_i2t + bias_t2t).reshape(1, IN))
    return out.reshape(N_NODES, H, F)
